# baseline (speedup 1.0000x reference)
import functools

import jax
import jax.numpy as jnp
from jax import lax
from jax.experimental import pallas as pl
from jax.experimental.pallas import tpu as pltpu
from jax.experimental.pallas import tpu_sc as plsc

CARDS_NUM = 1000000
EMB_DIM = 64
HIDDEN = 128
STATE = 64
BATCH = 4096
HIST = 200

N = BATCH * HIST
NW = 32
B_PER_W = N // NW
CH = 1024
N_CH = B_PER_W // CH


def _gather_body(table_hbm, idx_hbm, out_hbm, idx_v, rows_v, sem):
    core = lax.axis_index("c")
    sub = lax.axis_index("s")
    wid = sub * 2 + core
    base = wid * B_PER_W
    pltpu.sync_copy(idx_hbm.at[pl.ds(base, B_PER_W)], idx_v)

    def chunk(c, _):
        off = c * CH
        pltpu.async_copy(
            table_hbm.at[idx_v.at[pl.ds(off, CH)]], rows_v, sem
        ).wait()
        pltpu.sync_copy(rows_v, out_hbm.at[pl.ds(base + off, CH)])
        return ()

    lax.fori_loop(0, N_CH, chunk, (), unroll=False)


def _sc_gather(table, idx):
    mesh = plsc.VectorSubcoreMesh(core_axis_name="c", subcore_axis_name="s")
    return pl.kernel(
        _gather_body,
        out_type=jax.ShapeDtypeStruct((N, EMB_DIM), jnp.float32),
        mesh=mesh,
        scratch_types=[
            pltpu.VMEM((B_PER_W,), jnp.int32),
            pltpu.VMEM((CH, EMB_DIM), jnp.float32),
            pltpu.SemaphoreType.DMA,
        ],
        compiler_params=pltpu.CompilerParams(use_tc_tiling_on_sc=False),
    )(table, idx)


ROWS_BLK = 2048


def _mlp_body(x_ref, w1_ref, b1_ref, w2_ref, b2_ref, o_ref):
    x = x_ref[...]
    h = jax.nn.sigmoid(
        jnp.dot(x, w1_ref[...], preferred_element_type=jnp.float32)
        + b1_ref[...]
    )
    o_ref[...] = jax.nn.sigmoid(
        jnp.dot(h, w2_ref[...], preferred_element_type=jnp.float32)
        + b2_ref[...]
    )


def _tc_mlp(x, W1, b1, W2, b2):
    grid = (N // ROWS_BLK,)
    return pl.pallas_call(
        _mlp_body,
        grid=grid,
        in_specs=[
            pl.BlockSpec((ROWS_BLK, EMB_DIM), lambda i: (i, 0)),
            pl.BlockSpec((EMB_DIM, HIDDEN), lambda i: (0, 0)),
            pl.BlockSpec((1, HIDDEN), lambda i: (0, 0)),
            pl.BlockSpec((HIDDEN, STATE), lambda i: (0, 0)),
            pl.BlockSpec((1, STATE), lambda i: (0, 0)),
        ],
        out_specs=pl.BlockSpec((ROWS_BLK, STATE), lambda i: (i, 0)),
        out_shape=jax.ShapeDtypeStruct((N, STATE), jnp.float32),
    )(x, W1, b1, W2, b2)


@jax.jit
def kernel(cards_id, card_embedding, W1, b1, W2, b2):
    idx = cards_id.reshape(-1).astype(jnp.int32)
    gathered = _sc_gather(card_embedding, idx)
    out = _tc_mlp(gathered, W1, b1.reshape(1, HIDDEN), W2, b2.reshape(1, STATE))
    return out.reshape(BATCH, HIST, STATE)


# speedup vs baseline: 1.1243x; 1.1243x over previous
import functools

import jax
import jax.numpy as jnp
from jax import lax
from jax.experimental import pallas as pl
from jax.experimental.pallas import tpu as pltpu
from jax.experimental.pallas import tpu_sc as plsc

CARDS_NUM = 1000000
EMB_DIM = 64
HIDDEN = 128
STATE = 64
BATCH = 4096
HIST = 200

N = BATCH * HIST
NW = 32
B_PER_W = N // NW
CH = 1024
N_CH = B_PER_W // CH


def _gather_body(table_hbm, idx_hbm, out_hbm, idx_v, rows_v, sem):
    core = lax.axis_index("c")
    sub = lax.axis_index("s")
    wid = sub * 2 + core
    base = wid * B_PER_W
    pltpu.sync_copy(idx_hbm.at[pl.ds(base, B_PER_W)], idx_v)

    def chunk(c, _):
        off = c * CH
        pltpu.async_copy(
            table_hbm.at[idx_v.at[pl.ds(off, CH)]], rows_v, sem
        ).wait()
        pltpu.sync_copy(rows_v, out_hbm.at[pl.ds(base + off, CH)])
        return ()

    lax.fori_loop(0, N_CH, chunk, (), unroll=False)


def _sc_gather(table, idx):
    mesh = plsc.VectorSubcoreMesh(core_axis_name="c", subcore_axis_name="s")
    return pl.kernel(
        _gather_body,
        out_type=jax.ShapeDtypeStruct((N, EMB_DIM), jnp.float32),
        mesh=mesh,
        scratch_types=[
            pltpu.VMEM((B_PER_W,), jnp.int32),
            pltpu.VMEM((CH, EMB_DIM), jnp.float32),
            pltpu.SemaphoreType.DMA,
        ],
        compiler_params=pltpu.CompilerParams(use_tc_tiling_on_sc=False),
    )(table, idx)


ROWS_BLK = 8192


def _mlp_body(x_ref, w1_ref, b1_ref, w2_ref, b2_ref, o_ref):
    x = x_ref[...]
    h = jax.nn.sigmoid(
        jnp.dot(x, w1_ref[...], preferred_element_type=jnp.float32)
        + b1_ref[...]
    )
    o_ref[...] = jax.nn.sigmoid(
        jnp.dot(h, w2_ref[...], preferred_element_type=jnp.float32)
        + b2_ref[...]
    )


def _tc_mlp(x, W1, b1, W2, b2):
    grid = (N // ROWS_BLK,)
    return pl.pallas_call(
        _mlp_body,
        grid=grid,
        in_specs=[
            pl.BlockSpec((ROWS_BLK, EMB_DIM), lambda i: (i, 0)),
            pl.BlockSpec((EMB_DIM, HIDDEN), lambda i: (0, 0)),
            pl.BlockSpec((1, HIDDEN), lambda i: (0, 0)),
            pl.BlockSpec((HIDDEN, STATE), lambda i: (0, 0)),
            pl.BlockSpec((1, STATE), lambda i: (0, 0)),
        ],
        out_specs=pl.BlockSpec((ROWS_BLK, STATE), lambda i: (i, 0)),
        out_shape=jax.ShapeDtypeStruct((N, STATE), jnp.float32),
    )(x, W1, b1, W2, b2)


@jax.jit
def kernel(cards_id, card_embedding, W1, b1, W2, b2):
    idx = cards_id.reshape(-1).astype(jnp.int32)
    gathered = _sc_gather(card_embedding, idx)
    out = _tc_mlp(gathered, W1, b1.reshape(1, HIDDEN), W2, b2.reshape(1, STATE))
    return out.reshape(BATCH, HIST, STATE)


# speedup vs baseline: 1.3100x; 1.1652x over previous
import functools

import jax
import jax.numpy as jnp
from jax import lax
from jax.experimental import pallas as pl
from jax.experimental.pallas import tpu as pltpu
from jax.experimental.pallas import tpu_sc as plsc

CARDS_NUM = 1000000
HALF = CARDS_NUM // 2
EMB_DIM = 64
HIDDEN = 128
STATE = 64
BATCH = 4096
HIST = 200

N = BATCH * HIST
NW = 32
B_PER_W = N // NW
CH = 1024
N_CH = B_PER_W // CH
LANES = 16

ROWS_BLK = 5000


def _mlp2(x, w1, b1, w2, b2):
    h = jax.nn.sigmoid(jnp.dot(x, w1, preferred_element_type=jnp.float32) + b1)
    return jax.nn.sigmoid(jnp.dot(h, w2, preferred_element_type=jnp.float32) + b2)


def _table_mlp_body(lo_ref, hi_ref, w1_ref, b1_ref, w2_ref, b2_ref, o_ref):
    w1, b1, w2, b2 = w1_ref[...], b1_ref[...], w2_ref[...], b2_ref[...]
    o_ref[:, 0:STATE] = _mlp2(lo_ref[...], w1, b1, w2, b2)
    o_ref[:, STATE:2 * STATE] = _mlp2(hi_ref[...], w1, b1, w2, b2)


def _table_mlp(table, W1, b1, W2, b2):
    grid = (HALF // ROWS_BLK,)
    nblk = HALF // ROWS_BLK
    return pl.pallas_call(
        _table_mlp_body,
        grid=grid,
        in_specs=[
            pl.BlockSpec((ROWS_BLK, EMB_DIM), lambda i: (i, 0)),
            pl.BlockSpec((ROWS_BLK, EMB_DIM), lambda i, n=nblk: (i + n, 0)),
            pl.BlockSpec((EMB_DIM, HIDDEN), lambda i: (0, 0)),
            pl.BlockSpec((1, HIDDEN), lambda i: (0, 0)),
            pl.BlockSpec((HIDDEN, STATE), lambda i: (0, 0)),
            pl.BlockSpec((1, STATE), lambda i: (0, 0)),
        ],
        out_specs=pl.BlockSpec((ROWS_BLK, 2 * STATE), lambda i: (i, 0)),
        out_shape=jax.ShapeDtypeStruct((HALF, 2 * STATE), jnp.float32),
    )(table, table, W1, b1, W2, b2)


def _gather_body(table_hbm, idx_hbm, out_hbm, idx_v, rows_v, sem):
    core = lax.axis_index("c")
    sub = lax.axis_index("s")
    wid = sub * 2 + core
    base = wid * B_PER_W
    pltpu.sync_copy(idx_hbm.at[pl.ds(base, B_PER_W)], idx_v)

    def remap(j, _):
        v = idx_v[pl.ds(j * LANES, LANES)]
        ge = v >= HALF
        v2 = jnp.where(ge, 2 * (v - HALF) + 1, 2 * v)
        idx_v[pl.ds(j * LANES, LANES)] = v2
        return ()

    lax.fori_loop(0, B_PER_W // LANES, remap, (), unroll=8)

    def chunk(c, _):
        off = c * CH
        pltpu.async_copy(
            table_hbm.at[idx_v.at[pl.ds(off, CH)]], rows_v, sem
        ).wait()
        pltpu.sync_copy(rows_v, out_hbm.at[pl.ds(base + off, CH)])
        return ()

    lax.fori_loop(0, N_CH, chunk, (), unroll=False)


def _sc_gather(table, idx):
    mesh = plsc.VectorSubcoreMesh(core_axis_name="c", subcore_axis_name="s")
    return pl.kernel(
        _gather_body,
        out_type=jax.ShapeDtypeStruct((N, STATE), jnp.float32),
        mesh=mesh,
        scratch_types=[
            pltpu.VMEM((B_PER_W,), jnp.int32),
            pltpu.VMEM((CH, STATE), jnp.float32),
            pltpu.SemaphoreType.DMA,
        ],
        compiler_params=pltpu.CompilerParams(use_tc_tiling_on_sc=False),
    )(table, idx)


@jax.jit
def kernel(cards_id, card_embedding, W1, b1, W2, b2):
    idx = cards_id.reshape(-1).astype(jnp.int32)
    packed = _table_mlp(
        card_embedding, W1, b1.reshape(1, HIDDEN), W2, b2.reshape(1, STATE)
    )
    tview = packed.reshape(CARDS_NUM, STATE)
    out = _sc_gather(tview, idx)
    return out.reshape(BATCH, HIST, STATE)
